# Initial kernel scaffold; baseline (speedup 1.0000x reference)
import jax
import jax.numpy as jnp
from jax.experimental import pallas as pl


def kernel(hidden_states, kernel, e_score_correction_bias):
    raise NotImplementedError("write your pallas kernel here")


# speedup vs baseline: 3.4393x; 3.4393x over previous
import functools

import jax
import jax.numpy as jnp
from jax.experimental import pallas as pl

_NUM_TOKENS = 16384
_HIDDEN = 4096
_N_EXPERTS = 64
_TOP_K = 8
_N_GROUP = 8
_GROUP_SIZE = _N_EXPERTS // _N_GROUP
_TOPK_GROUP = 4
_SCALE = 2.5

_BT = 256
_NEG = -1e30


def _gate_kernel(h_ref, wt_ref, b_ref, o_ref):
    bt = h_ref.shape[0]
    logits_t = jax.lax.dot_general(
        wt_ref[...], h_ref[...],
        dimension_numbers=(((1,), (1,)), ((), ())),
        preferred_element_type=jnp.float32,
    )
    s = jax.nn.sigmoid(logits_t) + b_ref[...]
    sg = s.reshape(_N_GROUP, _GROUP_SIZE, bt)

    e_iota = jax.lax.broadcasted_iota(jnp.int32, sg.shape, 1)
    m1 = jnp.max(sg, axis=1, keepdims=True)
    p1 = jnp.min(jnp.where(sg == m1, e_iota, _GROUP_SIZE), axis=1, keepdims=True)
    m2 = jnp.max(jnp.where(e_iota == p1, _NEG, sg), axis=1, keepdims=True)
    gsum = m1 + m2

    g_iota = jax.lax.broadcasted_iota(jnp.int32, gsum.shape, 0)
    sel = jnp.zeros(gsum.shape, dtype=jnp.bool_)
    work = gsum
    for _ in range(_TOPK_GROUP):
        gmx = jnp.max(work, axis=0, keepdims=True)
        pg = jnp.min(jnp.where(work == gmx, g_iota, _N_GROUP), axis=0,
                     keepdims=True)
        hit = g_iota == pg
        sel = jnp.logical_or(sel, hit)
        work = jnp.where(hit, _NEG, work)

    cand = jnp.where(sel, sg, 0.0)
    pos = g_iota * _GROUP_SIZE + e_iota
    ws = []
    for _ in range(_TOP_K):
        cmx1 = jnp.max(cand, axis=1, keepdims=True)
        cmx = jnp.max(cmx1, axis=0, keepdims=True)
        pp = jnp.where(cand == cmx, pos, _N_EXPERTS)
        pmin = jnp.min(jnp.min(pp, axis=1, keepdims=True), axis=0,
                       keepdims=True)
        ws.append(cmx)
        cand = jnp.where(pos == pmin, _NEG, cand)

    wcat = jnp.concatenate([w.reshape(1, bt) for w in ws], axis=0)
    denom = jnp.sum(wcat, axis=0, keepdims=True) + 1e-20
    out_t = wcat / denom * _SCALE
    o_ref[...] = out_t.T


@functools.partial(jax.jit, static_argnames=())
def kernel(hidden_states, kernel, e_score_correction_bias):
    n_tokens = hidden_states.shape[0]
    wt = kernel.T
    b = e_score_correction_bias.reshape(_N_EXPERTS, 1)
    grid = (n_tokens // _BT,)
    out = pl.pallas_call(
        _gate_kernel,
        grid=grid,
        in_specs=[
            pl.BlockSpec((_BT, _HIDDEN), lambda i: (i, 0)),
            pl.BlockSpec((_N_EXPERTS, _HIDDEN), lambda i: (0, 0)),
            pl.BlockSpec((_N_EXPERTS, 1), lambda i: (0, 0)),
        ],
        out_specs=pl.BlockSpec((_BT, _TOP_K), lambda i: (i, 0)),
        out_shape=jax.ShapeDtypeStruct((n_tokens, _TOP_K), jnp.float32),
    )(hidden_states, wt, b)
    return out


# speedup vs baseline: 4.0732x; 1.1843x over previous
import functools

import jax
import jax.numpy as jnp
from jax.experimental import pallas as pl

_NUM_TOKENS = 16384
_HIDDEN = 4096
_N_EXPERTS = 64
_TOP_K = 8
_N_GROUP = 8
_GROUP_SIZE = _N_EXPERTS // _N_GROUP
_TOPK_GROUP = 4
_SCALE = 2.5

_BT = 512
_NEG = -1e30


def _gate_kernel(h_ref, wt_ref, b_ref, o_ref):
    bt = h_ref.shape[0]
    logits_t = jax.lax.dot_general(
        wt_ref[...], h_ref[...],
        dimension_numbers=(((1,), (1,)), ((), ())),
        preferred_element_type=jnp.float32,
    )
    s = jax.nn.sigmoid(logits_t) + b_ref[...]
    sg = s.reshape(_N_GROUP, _GROUP_SIZE, bt)

    e_iota = jax.lax.broadcasted_iota(jnp.int32, sg.shape, 1)
    m1 = jnp.max(sg, axis=1, keepdims=True)
    p1 = jnp.min(jnp.where(sg == m1, e_iota, _GROUP_SIZE), axis=1, keepdims=True)
    m2 = jnp.max(jnp.where(e_iota == p1, _NEG, sg), axis=1, keepdims=True)
    gsum = m1 + m2

    g_iota = jax.lax.broadcasted_iota(jnp.int32, gsum.shape, 0)
    sel = jnp.zeros(gsum.shape, dtype=jnp.bool_)
    work = gsum
    for _ in range(_TOPK_GROUP):
        gmx = jnp.max(work, axis=0, keepdims=True)
        pg = jnp.min(jnp.where(work == gmx, g_iota, _N_GROUP), axis=0,
                     keepdims=True)
        hit = g_iota == pg
        sel = jnp.logical_or(sel, hit)
        work = jnp.where(hit, _NEG, work)

    cand = jnp.where(sel, sg, 0.0)
    pos = g_iota * _GROUP_SIZE + e_iota
    ws = []
    for _ in range(_TOP_K):
        cmx1 = jnp.max(cand, axis=1, keepdims=True)
        cmx = jnp.max(cmx1, axis=0, keepdims=True)
        pp = jnp.where(cand == cmx, pos, _N_EXPERTS)
        pmin = jnp.min(jnp.min(pp, axis=1, keepdims=True), axis=0,
                       keepdims=True)
        ws.append(cmx)
        cand = jnp.where(pos == pmin, _NEG, cand)

    wcat = jnp.concatenate([w.reshape(1, bt) for w in ws], axis=0)
    denom = jnp.sum(wcat, axis=0, keepdims=True) + 1e-20
    out_t = wcat / denom * _SCALE
    o_ref[...] = out_t.T


@functools.partial(jax.jit, static_argnames=())
def kernel(hidden_states, kernel, e_score_correction_bias):
    n_tokens = hidden_states.shape[0]
    wt = kernel.T
    b = e_score_correction_bias.reshape(_N_EXPERTS, 1)
    grid = (n_tokens // _BT,)
    out = pl.pallas_call(
        _gate_kernel,
        grid=grid,
        in_specs=[
            pl.BlockSpec((_BT, _HIDDEN), lambda i: (i, 0)),
            pl.BlockSpec((_N_EXPERTS, _HIDDEN), lambda i: (0, 0)),
            pl.BlockSpec((_N_EXPERTS, 1), lambda i: (0, 0)),
        ],
        out_specs=pl.BlockSpec((_BT, _TOP_K), lambda i: (i, 0)),
        out_shape=jax.ShapeDtypeStruct((n_tokens, _TOP_K), jnp.float32),
    )(hidden_states, wt, b)
    return out


# speedup vs baseline: 4.5495x; 1.1169x over previous
import functools

import jax
import jax.numpy as jnp
from jax.experimental import pallas as pl

_NUM_TOKENS = 16384
_HIDDEN = 4096
_N_EXPERTS = 64
_TOP_K = 8
_N_GROUP = 8
_GROUP_SIZE = _N_EXPERTS // _N_GROUP
_TOPK_GROUP = 4
_SCALE = 2.5

_BT = 1024
_NEG = -1e30


def _gate_kernel(h_ref, wt_ref, b_ref, o_ref):
    bt = h_ref.shape[0]
    logits_t = jax.lax.dot_general(
        wt_ref[...], h_ref[...],
        dimension_numbers=(((1,), (1,)), ((), ())),
        preferred_element_type=jnp.float32,
    )
    s = jax.nn.sigmoid(logits_t) + b_ref[...]
    sg = s.reshape(_N_GROUP, _GROUP_SIZE, bt)

    e_iota = jax.lax.broadcasted_iota(jnp.int32, sg.shape, 1)
    m1 = jnp.max(sg, axis=1, keepdims=True)
    p1 = jnp.min(jnp.where(sg == m1, e_iota, _GROUP_SIZE), axis=1, keepdims=True)
    m2 = jnp.max(jnp.where(e_iota == p1, _NEG, sg), axis=1, keepdims=True)
    gsum = m1 + m2

    g_iota = jax.lax.broadcasted_iota(jnp.int32, gsum.shape, 0)
    sel = jnp.zeros(gsum.shape, dtype=jnp.bool_)
    work = gsum
    for _ in range(_TOPK_GROUP):
        gmx = jnp.max(work, axis=0, keepdims=True)
        pg = jnp.min(jnp.where(work == gmx, g_iota, _N_GROUP), axis=0,
                     keepdims=True)
        hit = g_iota == pg
        sel = jnp.logical_or(sel, hit)
        work = jnp.where(hit, _NEG, work)

    cand = jnp.where(sel, sg, 0.0)
    pos = g_iota * _GROUP_SIZE + e_iota
    ws = []
    for _ in range(_TOP_K):
        cmx1 = jnp.max(cand, axis=1, keepdims=True)
        cmx = jnp.max(cmx1, axis=0, keepdims=True)
        pp = jnp.where(cand == cmx, pos, _N_EXPERTS)
        pmin = jnp.min(jnp.min(pp, axis=1, keepdims=True), axis=0,
                       keepdims=True)
        ws.append(cmx)
        cand = jnp.where(pos == pmin, _NEG, cand)

    wcat = jnp.concatenate([w.reshape(1, bt) for w in ws], axis=0)
    denom = jnp.sum(wcat, axis=0, keepdims=True) + 1e-20
    out_t = wcat / denom * _SCALE
    o_ref[...] = out_t.T


@functools.partial(jax.jit, static_argnames=())
def kernel(hidden_states, kernel, e_score_correction_bias):
    n_tokens = hidden_states.shape[0]
    wt = kernel.T
    b = e_score_correction_bias.reshape(_N_EXPERTS, 1)
    grid = (n_tokens // _BT,)
    out = pl.pallas_call(
        _gate_kernel,
        grid=grid,
        in_specs=[
            pl.BlockSpec((_BT, _HIDDEN), lambda i: (i, 0)),
            pl.BlockSpec((_N_EXPERTS, _HIDDEN), lambda i: (0, 0)),
            pl.BlockSpec((_N_EXPERTS, 1), lambda i: (0, 0)),
        ],
        out_specs=pl.BlockSpec((_BT, _TOP_K), lambda i: (i, 0)),
        out_shape=jax.ShapeDtypeStruct((n_tokens, _TOP_K), jnp.float32),
    )(hidden_states, wt, b)
    return out
